# baseline (speedup 1.0000x reference)
import functools

import jax
import jax.numpy as jnp
from jax import lax
from jax.experimental import pallas as pl
from jax.experimental.pallas import tpu as pltpu
from jax.experimental.pallas import tpu_sc as plsc

_NC = 2
_NS = 16
_NW = _NC * _NS
_L = 16
_CHUNK = 16384
_HALF = _CHUNK // 2

_SIXTH = float(1.0 / 6.0)


def _make_sc_call(n, nknots):
    per_tile = n // _NW
    nchunks = per_tile // _CHUNK
    ci_max = nknots - 9
    mesh = plsc.VectorSubcoreMesh(
        core_axis_name="c", subcore_axis_name="s",
        num_cores=_NC, num_subcores=_NS)

    @functools.partial(
        pl.kernel,
        out_type=jax.ShapeDtypeStruct((n,), jnp.float32),
        mesh=mesh,
        compiler_params=pltpu.CompilerParams(needs_layout_passes=False),
        scratch_types=[
            pltpu.VMEM((2 * _CHUNK,), jnp.float32),
            pltpu.VMEM((2 * _CHUNK,), jnp.float32),
            pltpu.VMEM((_CHUNK,), jnp.float32),
            pltpu.VMEM((_CHUNK,), jnp.float32),
            pltpu.VMEM((48,), jnp.float32),
            pltpu.VMEM((16,), jnp.float32),
            pltpu.VMEM((32,), jnp.float32),
            pltpu.VMEM((32,), jnp.float32),
            pltpu.VMEM((32,), jnp.float32),
            pltpu.VMEM((32,), jnp.float32),
            pltpu.SemaphoreType.DMA,
            pltpu.SemaphoreType.DMA,
            pltpu.SemaphoreType.DMA,
            pltpu.SemaphoreType.DMA,
        ],
    )
    def run(x_hbm, cpad_hbm, consts_hbm, out_hbm,
            xb0, xb1, ob0, ob1, ctab, ktab, k0t, k1t, k2t, k3t,
            si0, si1, so0, so1):
        wid = lax.axis_index("s") * _NC + lax.axis_index("c")
        base = wid * per_tile

        pltpu.sync_copy(cpad_hbm, ctab)
        pltpu.sync_copy(consts_hbm, ktab)
        idx0 = jnp.zeros((_L,), jnp.int32)
        t0invhv = plsc.load_gather(ktab, [idx0])
        invhv = plsc.load_gather(ktab, [idx0 + 1])

        lanes = lax.iota(jnp.int32, _L)
        for j in range(2):
            idx = lanes + (_L * j)
            d0 = plsc.load_gather(ctab, [idx])
            d1 = plsc.load_gather(ctab, [idx + 1])
            d2 = plsc.load_gather(ctab, [idx + 2])
            d3 = plsc.load_gather(ctab, [idx + 3])
            sl = pl.ds(_L * j, _L)
            k0t[sl] = (d0 + 4.0 * d1 + d2) * _SIXTH
            k1t[sl] = (d2 - d0) * 0.5
            k2t[sl] = (d0 - 2.0 * d1 + d2) * 0.5
            k3t[sl] = (d3 - d0 + 3.0 * (d1 - d2)) * _SIXTH

        nbuf = 2
        xbufs = [xb0, xb1]
        obufs = [ob0, ob1]
        sins = [si0, si1]
        souts = [so0, so1]
        in_cp = [None] * nbuf
        out_cp = [None] * nbuf

        nin = nchunks // 2

        for ch in range(min(nbuf, nin)):
            in_cp[ch] = pltpu.async_copy(
                x_hbm.at[pl.ds(base + ch * 2 * _CHUNK, 2 * _CHUNK)],
                xbufs[ch], sins[ch])

        for ch in range(nin):
            bi = ch % nbuf
            in_cp[bi].wait()
            xb = xbufs[bi]
            for h in range(2):
                k = ch * 2 + h
                bo = k % nbuf
                if out_cp[bo] is not None:
                    out_cp[bo].wait()
                ob = obufs[bo]
                off = h * _CHUNK

                @plsc.parallel_loop(0, _CHUNK, _L, unroll=8)
                def _body(i):
                    xv = xb[pl.ds(i + off, _L)]
                    u = xv * invhv - t0invhv
                    g = u.astype(jnp.int32)
                    frac = u - g.astype(jnp.float32)
                    ci = jnp.minimum((g - 3).astype(jnp.uint32),
                                     jnp.uint32(ci_max)).astype(jnp.int32)
                    q0 = plsc.load_gather(k0t, [ci])
                    q1 = plsc.load_gather(k1t, [ci])
                    q2 = plsc.load_gather(k2t, [ci])
                    q3 = plsc.load_gather(k3t, [ci])
                    ob[pl.ds(i, _L)] = ((q3 * frac + q2) * frac + q1) * frac + q0

                out_cp[bo] = pltpu.async_copy(
                    ob, out_hbm.at[pl.ds(base + k * _CHUNK, _CHUNK)], souts[bo])
            nxt = ch + nbuf
            if nxt < nin:
                in_cp[bi] = pltpu.async_copy(
                    x_hbm.at[pl.ds(base + nxt * 2 * _CHUNK, 2 * _CHUNK)],
                    xbufs[bi], sins[bi])

        for cp in out_cp:
            if cp is not None:
                cp.wait()

    return run


def kernel(input, knots, control_points):
    x = input
    n = x.shape[0]
    nknots = knots.shape[0]
    t = jnp.sort(knots)
    t0 = t[0]
    invh = jnp.float32(nknots - 1) / (t[-1] - t[0])
    consts = jnp.zeros((16,), jnp.float32).at[0].set(t0 * invh).at[1].set(invh)
    cpad = jnp.zeros((48,), jnp.float32).at[: control_points.shape[0]].set(
        control_points)
    run = _make_sc_call(n, nknots)
    return run(x, cpad, consts)


# speedup vs baseline: 1.0116x; 1.0116x over previous
import functools

import jax
import jax.numpy as jnp
from jax import lax
from jax.experimental import pallas as pl
from jax.experimental.pallas import tpu as pltpu
from jax.experimental.pallas import tpu_sc as plsc

_NC = 2
_NS = 16
_NW = _NC * _NS
_L = 16
_CHUNK = 16384

_SIXTH = float(1.0 / 6.0)


def _make_sc_call(n, nknots):
    per_tile = n // _NW
    nchunks = per_tile // _CHUNK
    ci_max = nknots - 9
    mesh = plsc.VectorSubcoreMesh(
        core_axis_name="c", subcore_axis_name="s",
        num_cores=_NC, num_subcores=_NS)

    @functools.partial(
        pl.kernel,
        out_type=jax.ShapeDtypeStruct((n,), jnp.float32),
        mesh=mesh,
        compiler_params=pltpu.CompilerParams(needs_layout_passes=False),
        scratch_types=[
            pltpu.VMEM((_CHUNK,), jnp.float32),
            pltpu.VMEM((_CHUNK,), jnp.float32),
            pltpu.VMEM((_CHUNK,), jnp.float32),
            pltpu.VMEM((_CHUNK,), jnp.float32),
            pltpu.VMEM((48,), jnp.float32),
            pltpu.VMEM((16,), jnp.float32),
            pltpu.VMEM((32,), jnp.float32),
            pltpu.VMEM((32,), jnp.float32),
            pltpu.VMEM((32,), jnp.float32),
            pltpu.VMEM((32,), jnp.float32),
            pltpu.SemaphoreType.DMA,
            pltpu.SemaphoreType.DMA,
            pltpu.SemaphoreType.DMA,
            pltpu.SemaphoreType.DMA,
        ],
    )
    def run(x_hbm, cpad_hbm, consts_hbm, out_hbm,
            xb0, xb1, ob0, ob1, ctab, ktab, k0t, k1t, k2t, k3t,
            si0, si1, so0, so1):
        wid = lax.axis_index("s") * _NC + lax.axis_index("c")
        base = wid * per_tile

        pltpu.sync_copy(cpad_hbm, ctab)
        pltpu.sync_copy(consts_hbm, ktab)
        idx0 = jnp.zeros((_L,), jnp.int32)
        t0invhv = plsc.load_gather(ktab, [idx0])
        invhv = plsc.load_gather(ktab, [idx0 + 1])

        lanes = lax.iota(jnp.int32, _L)
        for j in range(2):
            idx = lanes + (_L * j)
            d0 = plsc.load_gather(ctab, [idx])
            d1 = plsc.load_gather(ctab, [idx + 1])
            d2 = plsc.load_gather(ctab, [idx + 2])
            d3 = plsc.load_gather(ctab, [idx + 3])
            sl = pl.ds(_L * j, _L)
            k0t[sl] = (d0 + 4.0 * d1 + d2) * _SIXTH
            k1t[sl] = (d2 - d0) * 0.5
            k2t[sl] = (d0 - 2.0 * d1 + d2) * 0.5
            k3t[sl] = (d3 - d0 + 3.0 * (d1 - d2)) * _SIXTH

        nbuf = 2
        xbufs = [xb0, xb1]
        obufs = [ob0, ob1]
        sins = [si0, si1]
        souts = [so0, so1]
        in_cp = [None] * nbuf
        out_cp = [None] * nbuf

        for ch in range(min(nbuf, nchunks)):
            in_cp[ch] = pltpu.async_copy(
                x_hbm.at[pl.ds(base + ch * _CHUNK, _CHUNK)], xbufs[ch], sins[ch])

        for ch in range(nchunks):
            b = ch % nbuf
            in_cp[b].wait()
            if out_cp[b] is not None:
                out_cp[b].wait()
            xb = xbufs[b]
            ob = obufs[b]

            @plsc.parallel_loop(0, _CHUNK, _L, unroll=8)
            def _body(i):
                xv = xb[pl.ds(i, _L)]
                u = xv * invhv - t0invhv
                g = u.astype(jnp.int32)
                frac = u - g.astype(jnp.float32)
                ci = jnp.minimum((g - 3).astype(jnp.uint32),
                                 jnp.uint32(ci_max)).astype(jnp.int32)
                q0 = plsc.load_gather(k0t, [ci])
                q1 = plsc.load_gather(k1t, [ci])
                q2 = plsc.load_gather(k2t, [ci])
                q3 = plsc.load_gather(k3t, [ci])
                ob[pl.ds(i, _L)] = ((q3 * frac + q2) * frac + q1) * frac + q0

            out_cp[b] = pltpu.async_copy(
                ob, out_hbm.at[pl.ds(base + ch * _CHUNK, _CHUNK)], souts[b])
            nxt = ch + nbuf
            if nxt < nchunks:
                in_cp[b] = pltpu.async_copy(
                    x_hbm.at[pl.ds(base + nxt * _CHUNK, _CHUNK)], xbufs[b], sins[b])

        for cp in out_cp:
            if cp is not None:
                cp.wait()

    return run


def kernel(input, knots, control_points):
    x = input
    n = x.shape[0]
    nknots = knots.shape[0]
    t = jnp.sort(knots)
    t0 = t[0]
    invh = jnp.float32(nknots - 1) / (t[-1] - t[0])
    consts = jnp.zeros((16,), jnp.float32).at[0].set(t0 * invh).at[1].set(invh)
    cpad = jnp.zeros((48,), jnp.float32).at[: control_points.shape[0]].set(
        control_points)
    run = _make_sc_call(n, nknots)
    return run(x, cpad, consts)
